# baseline (speedup 1.0000x reference)
import functools

import jax
import jax.numpy as jnp
from jax import lax
from jax.experimental import pallas as pl
from jax.experimental.pallas import tpu as pltpu
from jax.experimental.pallas import tpu_sc as plsc

B = 16384
EMB = 1000000
NEG = 20
D = 64
NC = 2
NS = 16
NW = NC * NS
BPW = B // NW
VCHUNK = 32
NVCH = BPW // VCHUNK
L = 16
NJ = D // L

IBLK = 16384
HBLK = IBLK // 2
LOG2H = HBLK.bit_length() - 1
TGRID = (EMB + IBLK - 1) // IBLK
EMBP = TGRID * IBLK

_MESH = dict(core_axis_name="c", subcore_axis_name="s")
_SC_PARAMS = pltpu.CompilerParams(use_tc_tiling_on_sc=False)


def _perm(v):
    return ((v & ~(IBLK - 1))
            | lax.shift_left(v & (HBLK - 1), 1)
            | (lax.shift_right_logical(v, LOG2H) & 1))


def _transpose_body(in_ref, out_ref):
    xt = in_ref[...].T
    out_ref[:, 0:D] = xt[0:HBLK]
    out_ref[:, D:2 * D] = xt[HBLK:IBLK]


def _tc_transpose(table_t):
    return pl.pallas_call(
        _transpose_body,
        grid=(TGRID,),
        in_specs=[pl.BlockSpec((D, IBLK), lambda g: (0, g))],
        out_specs=pl.BlockSpec((HBLK, 128), lambda g: (g, 0)),
        out_shape=jax.ShapeDtypeStruct((EMBP // 2, 128), jnp.float32),
    )(table_t)


def _sc_gather_u(u_lin, pos_u):
    mesh = plsc.VectorSubcoreMesh(**_MESH)

    @functools.partial(
        pl.kernel,
        out_type=jax.ShapeDtypeStruct((B, D), jnp.float32),
        mesh=mesh,
        scratch_types=[
            pltpu.VMEM((BPW,), jnp.int32),
            pltpu.VMEM((BPW,), jnp.int32),
            pltpu.VMEM((BPW, D), jnp.float32),
            pltpu.SemaphoreType.DMA,
        ],
        compiler_params=_SC_PARAMS,
    )
    def k(u_hbm, pu_hbm, out_hbm, idxu, idxu2, urows, sem):
        wid = lax.axis_index("s") * NC + lax.axis_index("c")
        base = wid * BPW
        pltpu.sync_copy(pu_hbm.at[pl.ds(base, BPW)], idxu)
        for g in range(BPW // L):
            v = idxu[pl.ds(g * L, L)]
            idxu2[pl.ds(g * L, L)] = _perm(v)
        pltpu.async_copy(u_hbm.at[idxu2], urows, sem).wait()
        pltpu.sync_copy(urows, out_hbm.at[pl.ds(base, BPW)])

    return k(u_lin, pos_u)


def _sc_gather_v(v_lin, pos_v, neg_v_flat):
    mesh = plsc.VectorSubcoreMesh(**_MESH)

    @functools.partial(
        pl.kernel,
        out_type=(
            jax.ShapeDtypeStruct((B, D), jnp.float32),
            jax.ShapeDtypeStruct((B, D), jnp.float32),
        ),
        mesh=mesh,
        scratch_types=[
            pltpu.VMEM((BPW,), jnp.int32),
            pltpu.VMEM((VCHUNK * NEG,), jnp.int32),
            pltpu.VMEM((BPW, D), jnp.float32),
            pltpu.VMEM((VCHUNK * NEG, D), jnp.float32),
            pltpu.VMEM((VCHUNK, D), jnp.float32),
            pltpu.SemaphoreType.DMA,
            pltpu.SemaphoreType.DMA,
        ],
        compiler_params=_SC_PARAMS,
    )
    def k(v_hbm, pv_hbm, nv_hbm, v_out, ns_out,
          idxv, idxn, vrows, nrows, nsum, sem_v, sem_n):
        wid = lax.axis_index("s") * NC + lax.axis_index("c")
        base = wid * BPW
        pltpu.sync_copy(pv_hbm.at[pl.ds(base, BPW)], idxv)
        for g in range(BPW // L):
            v = idxv[pl.ds(g * L, L)]
            idxv[pl.ds(g * L, L)] = _perm(v)
        cp_v = pltpu.async_copy(v_hbm.at[idxv], vrows, sem_v)

        def chunk_body(ci, _):
            cb = base + ci * VCHUNK
            pltpu.sync_copy(nv_hbm.at[pl.ds(cb * NEG, VCHUNK * NEG)], idxn)
            for g in range(VCHUNK * NEG // L):
                v = idxn[pl.ds(g * L, L)]
                idxn[pl.ds(g * L, L)] = _perm(v)
            pltpu.async_copy(v_hbm.at[idxn], nrows, sem_n).wait()

            def elem_body(e, _):
                ne = e * NEG
                acc = [None] * NJ
                for kk in range(NEG):
                    for j in range(NJ):
                        x = nrows[ne + kk, pl.ds(j * L, L)]
                        acc[j] = x if kk == 0 else acc[j] + x
                for j in range(NJ):
                    nsum[e, pl.ds(j * L, L)] = acc[j]
                return 0

            lax.fori_loop(0, VCHUNK, elem_body, 0)
            pltpu.sync_copy(nsum, ns_out.at[pl.ds(cb, VCHUNK)])
            return 0

        lax.fori_loop(0, NVCH, chunk_body, 0)
        cp_v.wait()
        pltpu.sync_copy(vrows, v_out.at[pl.ds(base, BPW)])

    return k(v_lin, pos_v, neg_v_flat)


def _loss_body(u_ref, v_ref, ns_ref, out_ref):
    rows = lax.broadcasted_iota(jnp.int32, (128, 2), 0)
    cols = lax.broadcasted_iota(jnp.int32, (128, 2), 1)
    m = ((rows // D) == cols).astype(jnp.float32)
    u = u_ref[...]
    sp = jnp.dot(u * v_ref[...], m, preferred_element_type=jnp.float32)
    sn = -jnp.dot(u * ns_ref[...], m, preferred_element_type=jnp.float32)
    lsp = jnp.minimum(sp, 0.0) - jnp.log1p(jnp.exp(-jnp.abs(sp)))
    lsn = jnp.minimum(sn, 0.0) - jnp.log1p(jnp.exp(-jnp.abs(sn)))
    out_ref[0, 0] = -(jnp.sum(lsp) + jnp.sum(lsn))


def _tc_loss(u_rows, v_rows, ns_rows):
    out = pl.pallas_call(
        _loss_body,
        out_shape=jax.ShapeDtypeStruct((1, 1), jnp.float32),
        out_specs=pl.BlockSpec(memory_space=pltpu.SMEM),
    )(u_rows, v_rows, ns_rows)
    return out[0, 0]


@jax.jit
def kernel(u_emb, v_emb, pos_u, pos_v, neg_v):
    v_lin = _tc_transpose(v_emb.T).reshape(EMBP, D)
    u_lin = _tc_transpose(u_emb.T).reshape(EMBP, D)
    v_rows, ns_rows = _sc_gather_v(v_lin, pos_v, neg_v.reshape(-1))
    u_rows = _sc_gather_u(u_lin, pos_u)
    return _tc_loss(
        u_rows.reshape(B // 2, 128),
        v_rows.reshape(B // 2, 128),
        ns_rows.reshape(B // 2, 128),
    )


# speedup vs baseline: 1.0580x; 1.0580x over previous
import functools

import jax
import jax.numpy as jnp
from jax import lax
from jax.experimental import pallas as pl
from jax.experimental.pallas import tpu as pltpu
from jax.experimental.pallas import tpu_sc as plsc

B = 16384
EMB = 1000000
NEG = 20
D = 64
NC = 2
NS = 16
NW = NC * NS
BPW = B // NW
VCHUNK = 32
NVCH = BPW // VCHUNK
L = 16
NJ = D // L

IBLK = 32768
HBLK = IBLK // 2
LOG2H = HBLK.bit_length() - 1
TGRID = (EMB + IBLK - 1) // IBLK
EMBP = TGRID * IBLK

_MESH = dict(core_axis_name="c", subcore_axis_name="s")
_SC_PARAMS = pltpu.CompilerParams(use_tc_tiling_on_sc=False)


def _perm(v):
    return ((v & ~(IBLK - 1))
            | lax.shift_left(v & (HBLK - 1), 1)
            | (lax.shift_right_logical(v, LOG2H) & 1))


def _transpose_body(in_ref, out_ref):
    xt = in_ref[...].T
    out_ref[:, 0:D] = xt[0:HBLK]
    out_ref[:, D:2 * D] = xt[HBLK:IBLK]


def _tc_transpose(table_t):
    return pl.pallas_call(
        _transpose_body,
        grid=(TGRID,),
        in_specs=[pl.BlockSpec((D, IBLK), lambda g: (0, g))],
        out_specs=pl.BlockSpec((HBLK, 128), lambda g: (g, 0)),
        out_shape=jax.ShapeDtypeStruct((EMBP // 2, 128), jnp.float32),
    )(table_t)


def _sc_gather_u(u_lin, pos_u):
    mesh = plsc.VectorSubcoreMesh(**_MESH)

    @functools.partial(
        pl.kernel,
        out_type=jax.ShapeDtypeStruct((B, D), jnp.float32),
        mesh=mesh,
        scratch_types=[
            pltpu.VMEM((BPW,), jnp.int32),
            pltpu.VMEM((BPW,), jnp.int32),
            pltpu.VMEM((BPW, D), jnp.float32),
            pltpu.SemaphoreType.DMA,
        ],
        compiler_params=_SC_PARAMS,
    )
    def k(u_hbm, pu_hbm, out_hbm, idxu, idxu2, urows, sem):
        wid = lax.axis_index("s") * NC + lax.axis_index("c")
        base = wid * BPW
        pltpu.sync_copy(pu_hbm.at[pl.ds(base, BPW)], idxu)
        for g in range(BPW // L):
            v = idxu[pl.ds(g * L, L)]
            idxu2[pl.ds(g * L, L)] = _perm(v)
        pltpu.async_copy(u_hbm.at[idxu2], urows, sem).wait()
        pltpu.sync_copy(urows, out_hbm.at[pl.ds(base, BPW)])

    return k(u_lin, pos_u)


def _sc_gather_v(v_lin, pos_v, neg_v_flat):
    mesh = plsc.VectorSubcoreMesh(**_MESH)

    @functools.partial(
        pl.kernel,
        out_type=(
            jax.ShapeDtypeStruct((B, D), jnp.float32),
            jax.ShapeDtypeStruct((B, D), jnp.float32),
        ),
        mesh=mesh,
        scratch_types=[
            pltpu.VMEM((BPW,), jnp.int32),
            pltpu.VMEM((VCHUNK * NEG,), jnp.int32),
            pltpu.VMEM((BPW, D), jnp.float32),
            pltpu.VMEM((VCHUNK * NEG, D), jnp.float32),
            pltpu.VMEM((VCHUNK, D), jnp.float32),
            pltpu.SemaphoreType.DMA,
            pltpu.SemaphoreType.DMA,
        ],
        compiler_params=_SC_PARAMS,
    )
    def k(v_hbm, pv_hbm, nv_hbm, v_out, ns_out,
          idxv, idxn, vrows, nrows, nsum, sem_v, sem_n):
        wid = lax.axis_index("s") * NC + lax.axis_index("c")
        base = wid * BPW
        pltpu.sync_copy(pv_hbm.at[pl.ds(base, BPW)], idxv)
        for g in range(BPW // L):
            v = idxv[pl.ds(g * L, L)]
            idxv[pl.ds(g * L, L)] = _perm(v)
        cp_v = pltpu.async_copy(v_hbm.at[idxv], vrows, sem_v)

        def chunk_body(ci, _):
            cb = base + ci * VCHUNK
            pltpu.sync_copy(nv_hbm.at[pl.ds(cb * NEG, VCHUNK * NEG)], idxn)
            for g in range(VCHUNK * NEG // L):
                v = idxn[pl.ds(g * L, L)]
                idxn[pl.ds(g * L, L)] = _perm(v)
            pltpu.async_copy(v_hbm.at[idxn], nrows, sem_n).wait()

            def elem_body(e, _):
                ne = e * NEG
                acc = [None] * NJ
                for kk in range(NEG):
                    for j in range(NJ):
                        x = nrows[ne + kk, pl.ds(j * L, L)]
                        acc[j] = x if kk == 0 else acc[j] + x
                for j in range(NJ):
                    nsum[e, pl.ds(j * L, L)] = acc[j]
                return 0

            lax.fori_loop(0, VCHUNK, elem_body, 0)
            pltpu.sync_copy(nsum, ns_out.at[pl.ds(cb, VCHUNK)])
            return 0

        lax.fori_loop(0, NVCH, chunk_body, 0)
        cp_v.wait()
        pltpu.sync_copy(vrows, v_out.at[pl.ds(base, BPW)])

    return k(v_lin, pos_v, neg_v_flat)


def _loss_body(u_ref, v_ref, ns_ref, out_ref):
    rows = lax.broadcasted_iota(jnp.int32, (128, 2), 0)
    cols = lax.broadcasted_iota(jnp.int32, (128, 2), 1)
    m = ((rows // D) == cols).astype(jnp.float32)
    u = u_ref[...]
    sp = jnp.dot(u * v_ref[...], m, preferred_element_type=jnp.float32)
    sn = -jnp.dot(u * ns_ref[...], m, preferred_element_type=jnp.float32)
    lsp = jnp.minimum(sp, 0.0) - jnp.log1p(jnp.exp(-jnp.abs(sp)))
    lsn = jnp.minimum(sn, 0.0) - jnp.log1p(jnp.exp(-jnp.abs(sn)))
    out_ref[0, 0] = -(jnp.sum(lsp) + jnp.sum(lsn))


def _tc_loss(u_rows, v_rows, ns_rows):
    out = pl.pallas_call(
        _loss_body,
        out_shape=jax.ShapeDtypeStruct((1, 1), jnp.float32),
        out_specs=pl.BlockSpec(memory_space=pltpu.SMEM),
    )(u_rows, v_rows, ns_rows)
    return out[0, 0]


@jax.jit
def kernel(u_emb, v_emb, pos_u, pos_v, neg_v):
    v_lin = _tc_transpose(v_emb.T).reshape(EMBP, D)
    u_lin = _tc_transpose(u_emb.T).reshape(EMBP, D)
    v_rows, ns_rows = _sc_gather_v(v_lin, pos_v, neg_v.reshape(-1))
    u_rows = _sc_gather_u(u_lin, pos_u)
    return _tc_loss(
        u_rows.reshape(B // 2, 128),
        v_rows.reshape(B // 2, 128),
        ns_rows.reshape(B // 2, 128),
    )


# speedup vs baseline: 1.0583x; 1.0002x over previous
import functools

import jax
import jax.numpy as jnp
from jax import lax
from jax.experimental import pallas as pl
from jax.experimental.pallas import tpu as pltpu
from jax.experimental.pallas import tpu_sc as plsc

B = 16384
EMB = 1000000
NEG = 20
D = 64
NC = 2
NS = 16
NW = NC * NS
BPW = B // NW
VCHUNK = 32
NVCH = BPW // VCHUNK
L = 16
NJ = D // L

IBLK = 32768
HBLK = IBLK // 2
LOG2H = HBLK.bit_length() - 1
TGRID = (EMB + IBLK - 1) // IBLK
EMBP = TGRID * IBLK

_MESH = dict(core_axis_name="c", subcore_axis_name="s")
_SC_PARAMS = pltpu.CompilerParams(use_tc_tiling_on_sc=False)


def _perm(v):
    return ((v & ~(IBLK - 1))
            | lax.shift_left(v & (HBLK - 1), 1)
            | (lax.shift_right_logical(v, LOG2H) & 1))


TCH = 2048


def _transpose_body(in_ref, out_ref):
    nch = IBLK // TCH
    for t in range(nch):
        xt = in_ref[:, t * TCH:(t + 1) * TCH].T
        if t < nch // 2:
            out_ref[t * TCH:(t + 1) * TCH, 0:D] = xt
        else:
            r = t - nch // 2
            out_ref[r * TCH:(r + 1) * TCH, D:2 * D] = xt


def _tc_transpose(table_t):
    return pl.pallas_call(
        _transpose_body,
        grid=(TGRID,),
        in_specs=[pl.BlockSpec((D, IBLK), lambda g: (0, g))],
        out_specs=pl.BlockSpec((HBLK, 128), lambda g: (g, 0)),
        out_shape=jax.ShapeDtypeStruct((EMBP // 2, 128), jnp.float32),
    )(table_t)


def _sc_gather_u(u_lin, pos_u):
    mesh = plsc.VectorSubcoreMesh(**_MESH)

    @functools.partial(
        pl.kernel,
        out_type=jax.ShapeDtypeStruct((B, D), jnp.float32),
        mesh=mesh,
        scratch_types=[
            pltpu.VMEM((BPW,), jnp.int32),
            pltpu.VMEM((BPW,), jnp.int32),
            pltpu.VMEM((BPW, D), jnp.float32),
            pltpu.SemaphoreType.DMA,
        ],
        compiler_params=_SC_PARAMS,
    )
    def k(u_hbm, pu_hbm, out_hbm, idxu, idxu2, urows, sem):
        wid = lax.axis_index("s") * NC + lax.axis_index("c")
        base = wid * BPW
        pltpu.sync_copy(pu_hbm.at[pl.ds(base, BPW)], idxu)
        for g in range(BPW // L):
            v = idxu[pl.ds(g * L, L)]
            idxu2[pl.ds(g * L, L)] = _perm(v)
        pltpu.async_copy(u_hbm.at[idxu2], urows, sem).wait()
        pltpu.sync_copy(urows, out_hbm.at[pl.ds(base, BPW)])

    return k(u_lin, pos_u)


def _sc_gather_v(v_lin, pos_v, neg_v_flat):
    mesh = plsc.VectorSubcoreMesh(**_MESH)

    @functools.partial(
        pl.kernel,
        out_type=(
            jax.ShapeDtypeStruct((B, D), jnp.float32),
            jax.ShapeDtypeStruct((B, D), jnp.float32),
        ),
        mesh=mesh,
        scratch_types=[
            pltpu.VMEM((BPW,), jnp.int32),
            pltpu.VMEM((VCHUNK * NEG,), jnp.int32),
            pltpu.VMEM((BPW, D), jnp.float32),
            pltpu.VMEM((VCHUNK * NEG, D), jnp.float32),
            pltpu.VMEM((VCHUNK, D), jnp.float32),
            pltpu.SemaphoreType.DMA,
            pltpu.SemaphoreType.DMA,
        ],
        compiler_params=_SC_PARAMS,
    )
    def k(v_hbm, pv_hbm, nv_hbm, v_out, ns_out,
          idxv, idxn, vrows, nrows, nsum, sem_v, sem_n):
        wid = lax.axis_index("s") * NC + lax.axis_index("c")
        base = wid * BPW
        pltpu.sync_copy(pv_hbm.at[pl.ds(base, BPW)], idxv)
        for g in range(BPW // L):
            v = idxv[pl.ds(g * L, L)]
            idxv[pl.ds(g * L, L)] = _perm(v)
        cp_v = pltpu.async_copy(v_hbm.at[idxv], vrows, sem_v)

        def chunk_body(ci, _):
            cb = base + ci * VCHUNK
            pltpu.sync_copy(nv_hbm.at[pl.ds(cb * NEG, VCHUNK * NEG)], idxn)
            for g in range(VCHUNK * NEG // L):
                v = idxn[pl.ds(g * L, L)]
                idxn[pl.ds(g * L, L)] = _perm(v)
            pltpu.async_copy(v_hbm.at[idxn], nrows, sem_n).wait()

            def elem_body(e, _):
                ne = e * NEG
                acc = [None] * NJ
                for kk in range(NEG):
                    for j in range(NJ):
                        x = nrows[ne + kk, pl.ds(j * L, L)]
                        acc[j] = x if kk == 0 else acc[j] + x
                for j in range(NJ):
                    nsum[e, pl.ds(j * L, L)] = acc[j]
                return 0

            lax.fori_loop(0, VCHUNK, elem_body, 0)
            pltpu.sync_copy(nsum, ns_out.at[pl.ds(cb, VCHUNK)])
            return 0

        lax.fori_loop(0, NVCH, chunk_body, 0)
        cp_v.wait()
        pltpu.sync_copy(vrows, v_out.at[pl.ds(base, BPW)])

    return k(v_lin, pos_v, neg_v_flat)


def _loss_body(u_ref, v_ref, ns_ref, out_ref):
    rows = lax.broadcasted_iota(jnp.int32, (128, 2), 0)
    cols = lax.broadcasted_iota(jnp.int32, (128, 2), 1)
    m = ((rows // D) == cols).astype(jnp.float32)
    u = u_ref[...]
    sp = jnp.dot(u * v_ref[...], m, preferred_element_type=jnp.float32)
    sn = -jnp.dot(u * ns_ref[...], m, preferred_element_type=jnp.float32)
    lsp = jnp.minimum(sp, 0.0) - jnp.log1p(jnp.exp(-jnp.abs(sp)))
    lsn = jnp.minimum(sn, 0.0) - jnp.log1p(jnp.exp(-jnp.abs(sn)))
    out_ref[0, 0] = -(jnp.sum(lsp) + jnp.sum(lsn))


def _tc_loss(u_rows, v_rows, ns_rows):
    out = pl.pallas_call(
        _loss_body,
        out_shape=jax.ShapeDtypeStruct((1, 1), jnp.float32),
        out_specs=pl.BlockSpec(memory_space=pltpu.SMEM),
    )(u_rows, v_rows, ns_rows)
    return out[0, 0]


@jax.jit
def kernel(u_emb, v_emb, pos_u, pos_v, neg_v):
    v_lin = _tc_transpose(v_emb.T).reshape(EMBP, D)
    u_lin = _tc_transpose(u_emb.T).reshape(EMBP, D)
    v_rows, ns_rows = _sc_gather_v(v_lin, pos_v, neg_v.reshape(-1))
    u_rows = _sc_gather_u(u_lin, pos_u)
    return _tc_loss(
        u_rows.reshape(B // 2, 128),
        v_rows.reshape(B // 2, 128),
        ns_rows.reshape(B // 2, 128),
    )
